# Initial kernel scaffold; baseline (speedup 1.0000x reference)
import jax
import jax.numpy as jnp
from jax.experimental import pallas as pl


def kernel(x, edge_index, edge_weight, Wz, bz, Wr, br, Wh, bh, W_lin, b_lin):
    raise NotImplementedError("write your pallas kernel here")


# speedup vs baseline: 20.3222x; 20.3222x over previous
import functools

import jax
import jax.numpy as jnp
from jax import lax
from jax.experimental import pallas as pl
from jax.experimental.pallas import tpu as pltpu
from jax.experimental.pallas import tpu_sc as plsc

N = 10000
E = 320000
IN_C = 128
F = 64
NC, NS = 2, 16
NW = NC * NS
CHUNK = 128
NPAD = 10240
EPAD = ((E + NW * CHUNK - 1) // (NW * CHUNK)) * (NW * CHUNK)
EPW = EPAD // NW
NCHUNKS = EPW // CHUNK
NPT = NPAD // NS

_MESH = plsc.VectorSubcoreMesh(core_axis_name="c", subcore_axis_name="s")
_SC_PARAMS = pltpu.CompilerParams(use_tc_tiling_on_sc=False)


def _deg_body(row_h, col_h, w_h, out_h, idx_v, val_v, buf_v, dego_sh, degi_sh):
    cid = lax.axis_index("c")
    sid = lax.axis_index("s")
    wid = sid * NC + cid

    def _zfill(i, _):
        buf_v[pl.ds(i * 16, 16)] = jnp.zeros((16,), jnp.float32)
        return 0
    lax.fori_loop(0, NPT // 16, _zfill, 0)
    pltpu.sync_copy(buf_v, dego_sh.at[pl.ds(sid * NPT, NPT)])
    pltpu.sync_copy(buf_v, degi_sh.at[pl.ds(sid * NPT, NPT)])
    plsc.subcore_barrier()

    def _body(j, _):
        base = wid * EPW + j * CHUNK
        pltpu.sync_copy(w_h.at[pl.ds(base, CHUNK)], val_v)
        pltpu.sync_copy(row_h.at[pl.ds(base, CHUNK)], idx_v)
        pltpu.sync_copy(val_v, dego_sh.at[idx_v], add=True)
        pltpu.sync_copy(col_h.at[pl.ds(base, CHUNK)], idx_v)
        pltpu.sync_copy(val_v, degi_sh.at[idx_v], add=True)
        return 0
    lax.fori_loop(0, NCHUNKS, _body, 0)
    plsc.subcore_barrier()

    pltpu.sync_copy(dego_sh.at[pl.ds(sid * NPT, NPT)], buf_v)
    pltpu.sync_copy(buf_v, out_h.at[cid, 0, pl.ds(sid * NPT, NPT)])
    pltpu.sync_copy(degi_sh.at[pl.ds(sid * NPT, NPT)], buf_v)
    pltpu.sync_copy(buf_v, out_h.at[cid, 1, pl.ds(sid * NPT, NPT)])


_deg_call = pl.kernel(
    _deg_body,
    out_type=jax.ShapeDtypeStruct((NC, 2, NPAD), jnp.float32),
    mesh=_MESH,
    compiler_params=_SC_PARAMS,
    scratch_types=[
        pltpu.VMEM((CHUNK,), jnp.int32),
        pltpu.VMEM((CHUNK,), jnp.float32),
        pltpu.VMEM((NPT,), jnp.float32),
        pltpu.VMEM_SHARED((NPAD,), jnp.float32),
        pltpu.VMEM_SHARED((NPAD,), jnp.float32),
    ],
)


def _spmm_body(row_h, col_h, yo_h, yi_h, z_h, out_h,
               idxr_v, idxc_v, gbo_v, gbi_v, acco_sh, acci_sh):
    cid = lax.axis_index("c")
    sid = lax.axis_index("s")
    wid = sid * NC + cid

    pltpu.sync_copy(z_h, gbo_v)

    def _zinit(t, _):
        rows = sid * NPT + t * CHUNK
        pltpu.sync_copy(gbo_v, acco_sh.at[pl.ds(rows, CHUNK)])
        pltpu.sync_copy(gbo_v, acci_sh.at[pl.ds(rows, CHUNK)])
        return 0
    lax.fori_loop(0, NPT // CHUNK, _zinit, 0)
    plsc.subcore_barrier()

    def _body(j, _):
        base = wid * EPW + j * CHUNK
        pltpu.sync_copy(row_h.at[pl.ds(base, CHUNK)], idxr_v)
        pltpu.sync_copy(col_h.at[pl.ds(base, CHUNK)], idxc_v)
        pltpu.sync_copy(yo_h.at[idxr_v], gbo_v)
        pltpu.sync_copy(gbo_v, acco_sh.at[idxc_v], add=True)
        pltpu.sync_copy(yi_h.at[idxc_v], gbi_v)
        pltpu.sync_copy(gbi_v, acci_sh.at[idxr_v], add=True)
        return 0
    lax.fori_loop(0, NCHUNKS, _body, 0)
    plsc.subcore_barrier()

    def _wout(t, _):
        rows = sid * NPT + t * CHUNK
        pltpu.sync_copy(acco_sh.at[pl.ds(rows, CHUNK)], gbo_v)
        pltpu.sync_copy(gbo_v, out_h.at[cid, 0, pl.ds(rows, CHUNK), :])
        pltpu.sync_copy(acci_sh.at[pl.ds(rows, CHUNK)], gbi_v)
        pltpu.sync_copy(gbi_v, out_h.at[cid, 1, pl.ds(rows, CHUNK), :])
        return 0
    lax.fori_loop(0, NPT // CHUNK, _wout, 0)


_spmm_call = pl.kernel(
    _spmm_body,
    out_type=jax.ShapeDtypeStruct((NC, 2, NPAD, F), jnp.float32),
    mesh=_MESH,
    compiler_params=_SC_PARAMS,
    scratch_types=[
        pltpu.VMEM((CHUNK,), jnp.int32),
        pltpu.VMEM((CHUNK,), jnp.int32),
        pltpu.VMEM((CHUNK, F), jnp.float32),
        pltpu.VMEM((CHUNK, F), jnp.float32),
        pltpu.VMEM_SHARED((NPAD, F), jnp.float32),
        pltpu.VMEM_SHARED((NPAD, F), jnp.float32),
    ],
)


def _prep_body(x_ref, deg_ref, wo_ref, wi_ref, w0_ref, yo_ref, yi_ref, a0_ref):
    deg = deg_ref[...]
    xb = x_ref[...]
    inv_o = (1.0 / (deg[0] + deg[2]))[:, None]
    inv_i = (1.0 / (deg[1] + deg[3]))[:, None]
    yo_ref[...] = jnp.dot(xb * inv_o, wo_ref[...], preferred_element_type=jnp.float32)
    yi_ref[...] = jnp.dot(xb * inv_i, wi_ref[...], preferred_element_type=jnp.float32)
    a0_ref[...] = jnp.dot(xb, w0_ref[...], preferred_element_type=jnp.float32)


def _combine_body(sp_ref, a0_ref, bzh_ref, wl_ref, bl_ref, out_ref):
    s = sp_ref[...]
    a = a0_ref[...] + s[0] + s[1] + s[2] + s[3] + bzh_ref[...]
    z = jax.nn.sigmoid(a[:, :32])
    ht = jnp.tanh(a[:, 32:])
    h = jax.nn.relu((1.0 - z) * ht)
    out_ref[...] = jnp.sum(h * wl_ref[...], axis=1, keepdims=True) + bl_ref[...]


def kernel(x, edge_index, edge_weight, Wz, bz, Wr, br, Wh, bh, W_lin, b_lin):
    row = edge_index[0]
    col = edge_index[1]
    pad_n = EPAD - E
    pad_idx = (jnp.arange(pad_n, dtype=jnp.int32) % (NPAD - N)) + N
    row_p = jnp.concatenate([row, pad_idx])
    col_p = jnp.concatenate([col, pad_idx])
    w_p = jnp.concatenate([edge_weight, jnp.zeros((pad_n,), jnp.float32)])
    x_pad = jnp.pad(x, ((0, NPAD - N), (0, 0)))

    Wo = jnp.concatenate([Wz[0, 1, :IN_C], Wh[0, 1, :IN_C]], axis=1)
    Wi = jnp.concatenate([Wz[1, 1, :IN_C], Wh[1, 1, :IN_C]], axis=1)
    W0 = jnp.concatenate([Wz[0, 0, :IN_C] + Wz[1, 0, :IN_C],
                          Wh[0, 0, :IN_C] + Wh[1, 0, :IN_C]], axis=1)
    bzh = jnp.concatenate([bz, bh]).reshape(1, F)
    wlT = W_lin.reshape(1, 32)
    bl = b_lin.reshape(1, 1)

    deg_part = _deg_call(row_p, col_p, w_p)
    deg4 = deg_part.reshape(4, NPAD)

    B = 512
    grid = (NPAD // B,)
    yo, yi, a0 = pl.pallas_call(
        _prep_body,
        grid=grid,
        in_specs=[
            pl.BlockSpec((B, IN_C), lambda i: (i, 0)),
            pl.BlockSpec((4, B), lambda i: (0, i)),
            pl.BlockSpec((IN_C, F), lambda i: (0, 0)),
            pl.BlockSpec((IN_C, F), lambda i: (0, 0)),
            pl.BlockSpec((IN_C, F), lambda i: (0, 0)),
        ],
        out_specs=[
            pl.BlockSpec((B, F), lambda i: (i, 0)),
            pl.BlockSpec((B, F), lambda i: (i, 0)),
            pl.BlockSpec((B, F), lambda i: (i, 0)),
        ],
        out_shape=[jax.ShapeDtypeStruct((NPAD, F), jnp.float32)] * 3,
    )(x_pad, deg4, Wo, Wi, W0)

    zeros_src = jnp.zeros((CHUNK, F), jnp.float32)
    s_part = _spmm_call(row_p, col_p, yo, yi, zeros_src)
    s4 = s_part.reshape(4, NPAD, F)

    out = pl.pallas_call(
        _combine_body,
        grid=grid,
        in_specs=[
            pl.BlockSpec((4, B, F), lambda i: (0, i, 0)),
            pl.BlockSpec((B, F), lambda i: (i, 0)),
            pl.BlockSpec((1, F), lambda i: (0, 0)),
            pl.BlockSpec((1, 32), lambda i: (0, 0)),
            pl.BlockSpec((1, 1), lambda i: (0, 0)),
        ],
        out_specs=pl.BlockSpec((B, 1), lambda i: (i, 0)),
        out_shape=jax.ShapeDtypeStruct((NPAD, 1), jnp.float32),
    )(s4, a0, bzh, wlT, bl)

    return out[:N]
